# baseline (speedup 1.0000x reference)
import functools

import jax
import jax.numpy as jnp
from jax import lax
from jax.experimental import pallas as pl
from jax.experimental.pallas import tpu as pltpu
from jax.experimental.pallas import tpu_sc as plsc

_C = 1.0
_EPS = 1e-5
_MIN_NORM = 1e-15
_PROJ_EPS = 4e-3

_NC = 2
_NS = 16
_NW = _NC * _NS

_CH = 128
_G = 8
_ZC = 128


def _logmap0_body(x_ref, o_ref):
    xb = x_ref[...]
    sq = jnp.sum(xb * xb, axis=-1, keepdims=True)
    norm = jnp.maximum(jnp.sqrt(sq), _MIN_NORM)
    arg = jnp.minimum(norm, 1.0 - _EPS)
    atanh = 0.5 * jnp.log((1.0 + arg) / (1.0 - arg))
    o_ref[...] = atanh * xb / norm


def _expmap_proj_body(p_ref, o_ref):
    u = p_ref[0] + p_ref[1]
    sq = jnp.sum(u * u, axis=-1, keepdims=True)
    norm = jnp.maximum(jnp.sqrt(sq), _MIN_NORM)
    y = jnp.tanh(norm) * u / norm
    sq2 = jnp.sum(y * y, axis=-1, keepdims=True)
    n2 = jnp.maximum(jnp.sqrt(sq2), _MIN_NORM)
    maxnorm = 1.0 - _PROJ_EPS
    o_ref[...] = jnp.where(n2 > maxnorm, y / n2 * maxnorm, y)


def _seg_sum_sc(n_pad, d, ngroups):
    rows_per_tile = n_pad // _NS
    nzero = rows_per_tile // _ZC
    assert rows_per_tile % _ZC == 0 and ngroups % 2 == 0

    mesh = plsc.VectorSubcoreMesh(core_axis_name="c", subcore_axis_name="s")

    @functools.partial(
        pl.kernel,
        out_type=jax.ShapeDtypeStruct((_NC, n_pad, d), jnp.float32),
        mesh=mesh,
        scratch_types=[
            pltpu.VMEM((_G, _CH), jnp.int32),
            pltpu.VMEM((_G, _CH), jnp.int32),
            pltpu.VMEM((_G, _CH), jnp.int32),
            pltpu.VMEM((_G, _CH), jnp.int32),
            pltpu.VMEM((_CH, d), jnp.float32),
            pltpu.VMEM((_CH, d), jnp.float32),
            pltpu.VMEM_SHARED((n_pad, d), jnp.float32),
            pltpu.SemaphoreType.DMA,
            pltpu.SemaphoreType.DMA,
            pltpu.SemaphoreType.DMA,
            pltpu.SemaphoreType.DMA,
        ],
    )
    def k(xt, s4, r4, zeros_hbm, out, sbA, rbA, sbB, rbB, rows0, rows1,
          accum, isemA, isemB, gsem0, gsem1):
        cid = lax.axis_index("c")
        sid = lax.axis_index("s")
        wid = cid * _NS + sid

        pltpu.sync_copy(zeros_hbm, rows0)
        row0 = sid * rows_per_tile
        for z in range(nzero):
            pltpu.sync_copy(rows0, accum.at[pl.ds(row0 + z * _ZC, _ZC)])
        plsc.subcore_barrier()

        rows_ = (rows0, rows1)
        gsem_ = (gsem0, gsem1)

        pltpu.async_copy(s4.at[wid, 0], sbA, isemA)
        pltpu.async_copy(r4.at[wid, 0], rbA, isemA)

        def process_group(g, sb, rb, isem, sb_n, rb_n, isem_n):
            pltpu.make_async_copy(s4.at[wid, 0], sb, isem).wait()
            pltpu.make_async_copy(r4.at[wid, 0], rb, isem).wait()

            @pl.when(g + 1 < ngroups)
            def _():
                pltpu.async_copy(s4.at[wid, g + 1], sb_n, isem_n)
                pltpu.async_copy(r4.at[wid, g + 1], rb_n, isem_n)

            pltpu.async_copy(xt.at[sb.at[0]], rows_[0], gsem_[0])
            for kk in range(_G):
                b = kk % 2
                if kk + 1 < _G:
                    pltpu.async_copy(xt.at[sb.at[kk + 1]], rows_[1 - b],
                                     gsem_[1 - b])
                pltpu.make_async_copy(xt.at[sb.at[kk]], rows_[b],
                                      gsem_[b]).wait()
                pltpu.sync_copy(rows_[b], accum.at[rb.at[kk]], add=True)

        def body(i, carry):
            g0 = 2 * i
            process_group(g0, sbA, rbA, isemA, sbB, rbB, isemB)
            process_group(g0 + 1, sbB, rbB, isemB, sbA, rbA, isemA)
            return carry

        lax.fori_loop(0, ngroups // 2, body, 0, unroll=False)
        plsc.subcore_barrier()

        for z in range(nzero):
            r = row0 + z * _ZC
            pltpu.sync_copy(accum.at[pl.ds(r, _ZC)], rows0)
            pltpu.sync_copy(rows0, out.at[cid, pl.ds(r, _ZC)])

    return k


def kernel(x, adj):
    n, d = x.shape
    e = adj.shape[1]
    n_pad = -(-n // (_NS * _ZC)) * (_NS * _ZC)
    assert d == 128

    bn = 1000
    x_t = pl.pallas_call(
        _logmap0_body,
        out_shape=jax.ShapeDtypeStruct((n, d), jnp.float32),
        grid=(n // bn,),
        in_specs=[pl.BlockSpec((bn, d), lambda i: (i, 0))],
        out_specs=pl.BlockSpec((bn, d), lambda i: (i, 0)),
    )(x)

    epw = -(-e // _NW)
    ngroups = max(2, -(-epw // (_CH * _G * 2)) * 2)
    e_pad = _NW * ngroups * _G * _CH
    s = adj[0]
    r = adj[1]
    if e_pad != e:
        s = jnp.concatenate([s, jnp.full((e_pad - e,), n, jnp.int32)])
        r = jnp.concatenate([r, jnp.zeros((e_pad - e,), jnp.int32)])
    s4 = s.reshape(_NW, ngroups, _G, _CH)
    r4 = r.reshape(_NW, ngroups, _G, _CH)
    xt_pad = jnp.concatenate([x_t, jnp.zeros((8, d), jnp.float32)])
    zeros = jnp.zeros((_ZC, d), jnp.float32)

    partials = _seg_sum_sc(n_pad, d, ngroups)(xt_pad, s4, r4, zeros)

    out = pl.pallas_call(
        _expmap_proj_body,
        out_shape=jax.ShapeDtypeStruct((n, d), jnp.float32),
        grid=(n // bn,),
        in_specs=[pl.BlockSpec((_NC, bn, d), lambda i: (0, i, 0))],
        out_specs=pl.BlockSpec((bn, d), lambda i: (i, 0)),
    )(partials)
    return out
